# baseline (speedup 1.0000x reference)
import functools

import jax
import jax.numpy as jnp
from jax import lax
from jax.experimental import pallas as pl
from jax.experimental.pallas import tpu as pltpu
from jax.experimental.pallas import tpu_sc as plsc

B, C, H, W, I = 8, 16, 512, 512, 4096
NC, NS, L = 2, 16, 16
NW = NC * NS
BC = B * C
BC_PER_W = BC // NW
HW = H * W
CH = I
UNROLL = 4
GROUPS = CH // (L * UNROLL)
NCHUNK = 2 * BC_PER_W


def _phys_addr(pbase, x, y):
    xhi = lax.shift_right_logical(x, 3)
    xlo = lax.bitwise_and(x, 7)
    yhi = lax.shift_right_logical(y, 7)
    ylo = lax.bitwise_and(y, 127)
    return pbase + xhi * 4096 + yhi * 1024 + xlo * 128 + ylo


def _loss_body(pred_hbm, iv0_hbm, iv1_hbm, nc0_hbm, nc1_hbm, out_hbm,
               bx0, by0, dx0, dy0, bx1, by1, dx1, dy1,
               pidx0, pidx1, pidx2, pv0, pv1, pv2,
               nc0buf, nc1buf, accbuf,
               semf0, semf1, semg0, semg1, semg2):
    cid = lax.axis_index("c")
    sid = lax.axis_index("s")
    wid = sid * NC + cid

    iota = lax.iota(jnp.int32, L)

    pltpu.sync_copy(nc0_hbm, nc0buf)
    pltpu.sync_copy(nc1_hbm, nc1buf)

    fields = ((bx0, by0, dx0, dy0), (bx1, by1, dx1, dy1))
    idxs = (pidx0, pidx1, pidx2)
    vals = (pv0, pv1, pv2)
    semf = (semf0, semf1)
    semg = (semg0, semg1, semg2)

    ivs = (iv0_hbm, iv0_hbm, iv0_hbm, iv0_hbm,
           iv1_hbm, iv1_hbm, iv1_hbm, iv1_hbm)
    ncbufs = (nc0buf, nc0buf, nc0buf, nc0buf,
              nc1buf, nc1buf, nc1buf, nc1buf)

    def chunk_bc(t):
        return wid * BC_PER_W + (t % BC_PER_W)

    def start_fields(t):
        bc = chunk_bc(t)
        b = lax.div(bc, C)
        c = lax.rem(bc, C)
        iv = ivs[t]
        fb = fields[t % 2]
        sem = semf[t % 2]
        return [pltpu.async_copy(iv.at[b, c, p, x], fb[2 * p + x], sem)
                for p in range(2) for x in range(2)]

    def build_indices(t):
        bc = chunk_bc(t)
        pbase = bc * HW
        fbx, fby, fdx, fdy = fields[t % 2]
        pidx = idxs[t % 3]

        def body(j, carry):
            for u in range(UNROLL):
                sl = pl.ds((j * UNROLL + u) * L, L)
                dsl = pl.ds(CH + (j * UNROLL + u) * L, L)
                pidx[sl] = _phys_addr(carry, fbx[sl], fby[sl])
                pidx[dsl] = _phys_addr(carry, fdx[sl], fdy[sl])
            return carry

        lax.fori_loop(0, GROUPS, body, pbase)

    def start_gathers(t):
        pidx = idxs[t % 3]
        pv = vals[t % 3]
        sem = semg[t % 3]
        return [pltpu.async_copy(pred_hbm.at[pidx], pv, sem)]

    def accumulate(t, acc):
        bc = chunk_bc(t)
        c = lax.rem(bc, C)
        ncbuf = ncbufs[t]
        t_vec = plsc.load_gather(ncbuf, [jnp.full((L,), c, jnp.int32)])
        t_vec = jnp.minimum(t_vec, I)
        pv = vals[t % 3]

        def body(j, a):
            for u in range(UNROLL):
                g = j * UNROLL + u
                sl = pl.ds(g * L, L)
                dsl = pl.ds(CH + g * L, L)
                d = pv[sl] - pv[dsl]
                d2 = d * d
                good = (g * L + iota) < t_vec
                a = a + jnp.where(good, 1.0 - d2, d2)
            return a

        return lax.fori_loop(0, GROUPS, body, acc)

    acc = jnp.zeros((L,), jnp.float32)
    f_cps = start_fields(0)
    g_pend = []
    for t in range(NCHUNK):
        for cp in f_cps:
            cp.wait()
        if t + 1 < NCHUNK:
            f_cps = start_fields(t + 1)
        build_indices(t)
        g_pend.append((t, start_gathers(t)))
        if len(g_pend) == 3:
            tg, cps = g_pend.pop(0)
            for cp in cps:
                cp.wait()
            acc = accumulate(tg, acc)
    for tg, cps in g_pend:
        for cp in cps:
            cp.wait()
        acc = accumulate(tg, acc)

    accbuf[...] = acc
    pltpu.sync_copy(accbuf, out_hbm.at[wid])


@jax.jit
def _loss(pred_flat, iv0_t, iv1_t, nc0, nc1):
    mesh = plsc.VectorSubcoreMesh(core_axis_name="c", subcore_axis_name="s")
    run = pl.kernel(
        _loss_body,
        out_type=jax.ShapeDtypeStruct((NW, L), jnp.float32),
        mesh=mesh,
        compiler_params=pltpu.CompilerParams(needs_layout_passes=False),
        scratch_types=(
            [pltpu.VMEM((CH,), jnp.int32) for _ in range(8)]
            + [pltpu.VMEM((2 * CH,), jnp.int32) for _ in range(3)]
            + [pltpu.VMEM((2 * CH,), jnp.float32) for _ in range(3)]
            + [pltpu.VMEM((L,), jnp.int32),
               pltpu.VMEM((L,), jnp.int32),
               pltpu.VMEM((L,), jnp.float32),
               pltpu.SemaphoreType.DMA,
               pltpu.SemaphoreType.DMA,
               pltpu.SemaphoreType.DMA,
               pltpu.SemaphoreType.DMA,
               pltpu.SemaphoreType.DMA]
        ),
    )
    partials = run(pred_flat, iv0_t, iv1_t, nc0, nc1)
    return jnp.sum(partials)


def kernel(prediction, intervals_comp_0, intervals_comp_1,
           good_intervals_0, good_intervals_1):
    pred_flat = (prediction.reshape(B, C, H // 8, 8, W // 128, 128)
                 .transpose(0, 1, 2, 4, 3, 5).reshape(-1))
    iv0_t = intervals_comp_0.transpose(0, 1, 3, 4, 2)
    iv1_t = intervals_comp_1.transpose(0, 1, 3, 4, 2)
    return _loss(pred_flat, iv0_t, iv1_t,
                 good_intervals_0, good_intervals_1)


# speedup vs baseline: 1.0298x; 1.0298x over previous
import functools

import jax
import jax.numpy as jnp
from jax import lax
from jax.experimental import pallas as pl
from jax.experimental.pallas import tpu as pltpu
from jax.experimental.pallas import tpu_sc as plsc

B, C, H, W, I = 8, 16, 512, 512, 4096
NC, NS, L = 2, 16, 16
NW = NC * NS
BC = B * C
BC_PER_W = BC // NW
HW = H * W
CH = I
UNROLL = 4
GROUPS = CH // (L * UNROLL)
NCHUNK = 2 * BC_PER_W


def _phys_addr(pbase, x, y):
    xhi = lax.shift_right_logical(x, 3)
    xlo = lax.bitwise_and(x, 7)
    yhi = lax.shift_right_logical(y, 7)
    ylo = lax.bitwise_and(y, 127)
    return pbase + xhi * 4096 + yhi * 1024 + xlo * 128 + ylo


def _loss_body(pred_hbm, iv0_hbm, iv1_hbm, nc0_hbm, nc1_hbm, out_hbm,
               bx0, by0, dx0, dy0, bx1, by1, dx1, dy1,
               bidx0, didx0, bidx1, didx1, bidx2, didx2,
               bvals0, dvals0, bvals1, dvals1, bvals2, dvals2,
               nc0buf, nc1buf, accbuf,
               semf0, semf1, semg0, semg1, semg2):
    cid = lax.axis_index("c")
    sid = lax.axis_index("s")
    wid = sid * NC + cid

    iota = lax.iota(jnp.int32, L)

    pltpu.sync_copy(nc0_hbm, nc0buf)
    pltpu.sync_copy(nc1_hbm, nc1buf)

    fields = ((bx0, by0, dx0, dy0), (bx1, by1, dx1, dy1))
    idxs = ((bidx0, didx0), (bidx1, didx1), (bidx2, didx2))
    vals = ((bvals0, dvals0), (bvals1, dvals1), (bvals2, dvals2))
    semf = (semf0, semf1)
    semg = (semg0, semg1, semg2)

    ivs = (iv0_hbm, iv0_hbm, iv0_hbm, iv0_hbm,
           iv1_hbm, iv1_hbm, iv1_hbm, iv1_hbm)
    ncbufs = (nc0buf, nc0buf, nc0buf, nc0buf,
              nc1buf, nc1buf, nc1buf, nc1buf)

    def chunk_bc(t):
        return wid * BC_PER_W + (t % BC_PER_W)

    def start_fields(t):
        bc = chunk_bc(t)
        b = lax.div(bc, C)
        c = lax.rem(bc, C)
        iv = ivs[t]
        fb = fields[t % 2]
        sem = semf[t % 2]
        return [pltpu.async_copy(iv.at[b, c, p, x], fb[2 * p + x], sem)
                for p in range(2) for x in range(2)]

    def build_indices(t):
        bc = chunk_bc(t)
        pbase = bc * HW
        fbx, fby, fdx, fdy = fields[t % 2]
        bidx, didx = idxs[t % 3]

        def body(j, carry):
            for u in range(UNROLL):
                sl = pl.ds((j * UNROLL + u) * L, L)
                bidx[sl] = _phys_addr(carry, fbx[sl], fby[sl])
                didx[sl] = _phys_addr(carry, fdx[sl], fdy[sl])
            return carry

        lax.fori_loop(0, GROUPS, body, pbase)

    def start_gathers(t):
        bidx, didx = idxs[t % 3]
        bv, dv = vals[t % 3]
        sem = semg[t % 3]
        return [pltpu.async_copy(pred_hbm.at[bidx], bv, sem),
                pltpu.async_copy(pred_hbm.at[didx], dv, sem)]

    def accumulate(t, acc):
        bc = chunk_bc(t)
        c = lax.rem(bc, C)
        ncbuf = ncbufs[t]
        t_vec = plsc.load_gather(ncbuf, [jnp.full((L,), c, jnp.int32)])
        t_vec = jnp.minimum(t_vec, I)
        bv, dv = vals[t % 3]

        def body(j, a):
            for u in range(UNROLL):
                g = j * UNROLL + u
                sl = pl.ds(g * L, L)
                d = bv[sl] - dv[sl]
                d2 = d * d
                good = (g * L + iota) < t_vec
                a = a + jnp.where(good, 1.0 - d2, d2)
            return a

        return lax.fori_loop(0, GROUPS, body, acc)

    acc = jnp.zeros((L,), jnp.float32)
    f_cps = start_fields(0)
    g_pend = []
    for t in range(NCHUNK):
        for cp in f_cps:
            cp.wait()
        if t + 1 < NCHUNK:
            f_cps = start_fields(t + 1)
        build_indices(t)
        g_pend.append((t, start_gathers(t)))
        if len(g_pend) == 3:
            tg, cps = g_pend.pop(0)
            for cp in cps:
                cp.wait()
            acc = accumulate(tg, acc)
    for tg, cps in g_pend:
        for cp in cps:
            cp.wait()
        acc = accumulate(tg, acc)

    accbuf[...] = acc
    pltpu.sync_copy(accbuf, out_hbm.at[wid])


@jax.jit
def _loss(pred_flat, iv0_t, iv1_t, nc0, nc1):
    mesh = plsc.VectorSubcoreMesh(core_axis_name="c", subcore_axis_name="s")
    run = pl.kernel(
        _loss_body,
        out_type=jax.ShapeDtypeStruct((NW, L), jnp.float32),
        mesh=mesh,
        compiler_params=pltpu.CompilerParams(needs_layout_passes=False),
        scratch_types=(
            [pltpu.VMEM((CH,), jnp.int32) for _ in range(8)]
            + [pltpu.VMEM((CH,), jnp.int32) for _ in range(6)]
            + [pltpu.VMEM((CH,), jnp.float32) for _ in range(6)]
            + [pltpu.VMEM((L,), jnp.int32),
               pltpu.VMEM((L,), jnp.int32),
               pltpu.VMEM((L,), jnp.float32),
               pltpu.SemaphoreType.DMA,
               pltpu.SemaphoreType.DMA,
               pltpu.SemaphoreType.DMA,
               pltpu.SemaphoreType.DMA,
               pltpu.SemaphoreType.DMA]
        ),
    )
    partials = run(pred_flat, iv0_t, iv1_t, nc0, nc1)
    return jnp.sum(partials)


def kernel(prediction, intervals_comp_0, intervals_comp_1,
           good_intervals_0, good_intervals_1):
    pred_flat = (prediction.reshape(B, C, H // 8, 8, W // 128, 128)
                 .transpose(0, 1, 2, 4, 3, 5).reshape(-1))
    iv0_t = intervals_comp_0.transpose(0, 1, 3, 4, 2)
    iv1_t = intervals_comp_1.transpose(0, 1, 3, 4, 2)
    return _loss(pred_flat, iv0_t, iv1_t,
                 good_intervals_0, good_intervals_1)


# speedup vs baseline: 1.0306x; 1.0008x over previous
import jax
import jax.numpy as jnp
from jax import lax
from jax.experimental import pallas as pl
from jax.experimental.pallas import tpu as pltpu
from jax.experimental.pallas import tpu_sc as plsc

B, C, H, W, I = 8, 16, 512, 512, 4096
NC, NS, L = 2, 16, 16
NW = NC * NS
BC = B * C
BC_PER_W = BC // NW
HW = H * W
CH = I
UNROLL = 4
GROUPS = CH // (L * UNROLL)
NCHUNK = 2 * BC_PER_W


def _phys_addr(pbase, x, y):
    xhi = lax.shift_right_logical(x, 3)
    xlo = lax.bitwise_and(x, 7)
    yhi = lax.shift_right_logical(y, 7)
    ylo = lax.bitwise_and(y, 127)
    return pbase + xhi * 4096 + yhi * 1024 + xlo * 128 + ylo


def _loss_body(pred_hbm, iv0_hbm, iv1_hbm, nc0_hbm, nc1_hbm, out_hbm,
               bx0, by0, dx0, dy0, bx1, by1, dx1, dy1,
               bidx0, didx0, bidx1, didx1, bidx2, didx2,
               bvals0, dvals0, bvals1, dvals1, bvals2, dvals2,
               nc0buf, nc1buf, accbuf,
               semf0, semf1, semg0, semg1, semg2):
    cid = lax.axis_index("c")
    sid = lax.axis_index("s")
    wid = sid * NC + cid

    iota = lax.iota(jnp.int32, L)

    pltpu.sync_copy(nc0_hbm, nc0buf)
    pltpu.sync_copy(nc1_hbm, nc1buf)

    fields = ((bx0, by0, dx0, dy0), (bx1, by1, dx1, dy1))
    idxs = ((bidx0, didx0), (bidx1, didx1), (bidx2, didx2))
    vals = ((bvals0, dvals0), (bvals1, dvals1), (bvals2, dvals2))
    semf = (semf0, semf1)
    semg = (semg0, semg1, semg2)

    ivs = (iv0_hbm, iv0_hbm, iv0_hbm, iv0_hbm,
           iv1_hbm, iv1_hbm, iv1_hbm, iv1_hbm)
    ncbufs = (nc0buf, nc0buf, nc0buf, nc0buf,
              nc1buf, nc1buf, nc1buf, nc1buf)

    def chunk_bc(t):
        return wid * BC_PER_W + (t % BC_PER_W)

    def start_fields(t):
        bc = chunk_bc(t)
        b = lax.div(bc, C)
        c = lax.rem(bc, C)
        iv = ivs[t]
        fb = fields[t % 2]
        sem = semf[t % 2]
        return [pltpu.async_copy(iv.at[b, c, p, x], fb[2 * p + x], sem)
                for p in range(2) for x in range(2)]

    def build_indices(t):
        bc = chunk_bc(t)
        pbase = bc * HW
        fbx, fby, fdx, fdy = fields[t % 2]
        bidx, didx = idxs[t % 3]

        def body(j, carry):
            for u in range(UNROLL):
                sl = pl.ds((j * UNROLL + u) * L, L)
                bidx[sl] = _phys_addr(carry, fbx[sl], fby[sl])
                didx[sl] = _phys_addr(carry, fdx[sl], fdy[sl])
            return carry

        lax.fori_loop(0, GROUPS, body, pbase)

    def start_gathers(t):
        bidx, didx = idxs[t % 3]
        bv, dv = vals[t % 3]
        sem = semg[t % 3]
        return [pltpu.async_copy(pred_hbm.at[bidx], bv, sem),
                pltpu.async_copy(pred_hbm.at[didx], dv, sem)]

    def accumulate(t, acc):
        bc = chunk_bc(t)
        c = lax.rem(bc, C)
        ncbuf = ncbufs[t]
        t_vec = plsc.load_gather(ncbuf, [jnp.full((L,), c, jnp.int32)])
        t_vec = jnp.minimum(t_vec, I)
        bv, dv = vals[t % 3]

        def body(j, a):
            for u in range(UNROLL):
                g = j * UNROLL + u
                sl = pl.ds(g * L, L)
                d = bv[sl] - dv[sl]
                d2 = d * d
                good = (g * L + iota) < t_vec
                a = a + jnp.where(good, 1.0 - d2, d2)
            return a

        return lax.fori_loop(0, GROUPS, body, acc)

    acc = jnp.zeros((L,), jnp.float32)
    f_cps = start_fields(0)
    g_pend = []
    for t in range(NCHUNK):
        for cp in f_cps:
            cp.wait()
        if t + 1 < NCHUNK:
            f_cps = start_fields(t + 1)
        build_indices(t)
        g_pend.append((t, start_gathers(t)))
        if len(g_pend) == 3:
            tg, cps = g_pend.pop(0)
            for cp in cps:
                cp.wait()
            acc = accumulate(tg, acc)
    for tg, cps in g_pend:
        for cp in cps:
            cp.wait()
        acc = accumulate(tg, acc)

    accbuf[...] = acc
    pltpu.sync_copy(accbuf, out_hbm.at[wid])


@jax.jit
def _loss(pred_flat, iv0_t, iv1_t, nc0, nc1):
    mesh = plsc.VectorSubcoreMesh(core_axis_name="c", subcore_axis_name="s")
    run = pl.kernel(
        _loss_body,
        out_type=jax.ShapeDtypeStruct((NW, L), jnp.float32),
        mesh=mesh,
        compiler_params=pltpu.CompilerParams(needs_layout_passes=False),
        scratch_types=(
            [pltpu.VMEM((CH,), jnp.int32) for _ in range(8)]
            + [pltpu.VMEM((CH,), jnp.int32) for _ in range(6)]
            + [pltpu.VMEM((CH,), jnp.float32) for _ in range(6)]
            + [pltpu.VMEM((L,), jnp.int32),
               pltpu.VMEM((L,), jnp.int32),
               pltpu.VMEM((L,), jnp.float32),
               pltpu.SemaphoreType.DMA,
               pltpu.SemaphoreType.DMA,
               pltpu.SemaphoreType.DMA,
               pltpu.SemaphoreType.DMA,
               pltpu.SemaphoreType.DMA]
        ),
    )
    partials = run(pred_flat, iv0_t, iv1_t, nc0, nc1)
    return jnp.sum(partials)


def kernel(prediction, intervals_comp_0, intervals_comp_1,
           good_intervals_0, good_intervals_1):
    pred_flat = (prediction.reshape(B, C, H // 8, 8, W // 128, 128)
                 .transpose(0, 1, 2, 4, 3, 5).reshape(-1))
    iv0_t = intervals_comp_0.transpose(0, 1, 3, 4, 2)
    iv1_t = intervals_comp_1.transpose(0, 1, 3, 4, 2)
    return _loss(pred_flat, iv0_t, iv1_t,
                 good_intervals_0, good_intervals_1)
